# baseline (speedup 1.0000x reference)
import functools

import jax
import jax.numpy as jnp
from jax import lax
from jax.experimental import pallas as pl
from jax.experimental.pallas import tpu as pltpu
from jax.experimental.pallas import tpu_sc as plsc

D = 768
H = 12
HD = 64
FF = 2048
E = 8
T = 2048
EPS = 1e-8
LN_EPS = 1e-5

TQ = 256
TA = 256
BM = 256
NS = T * 2 + E * BM
NB = NS // BM
NW = 32
CHUNK = T // NW


def _ln(x, g, b):
    m = jnp.mean(x, axis=-1, keepdims=True)
    v = jnp.mean((x - m) ** 2, axis=-1, keepdims=True)
    return (x - m) * jax.lax.rsqrt(v + LN_EPS) * g + b


def _qkv_body(s_ref, w_ref, b_ref, o_ref):
    o = jnp.dot(s_ref[...], w_ref[...],
                preferred_element_type=jnp.float32) + b_ref[...]
    o_ref[...] = o.reshape(T, 4, HD).transpose(1, 0, 2)


def _qkv(src2, w3, b3):
    BN = 256
    return pl.pallas_call(
        _qkv_body,
        grid=(3 * D // BN,),
        in_specs=[
            pl.BlockSpec((T, D), lambda n: (0, 0)),
            pl.BlockSpec((D, BN), lambda n: (0, n)),
            pl.BlockSpec((1, BN), lambda n: (0, n)),
        ],
        out_specs=pl.BlockSpec((4, T, HD), lambda n: (n, 0, 0)),
        out_shape=jax.ShapeDtypeStruct((3 * H, T, HD), jnp.float32),
    )(src2, w3, b3)


def _attn_body(q_ref, k_ref, v_ref, fi_ref, fj_ref, ab_ref, o_ref, fac_ref):
    h = pl.program_id(1)
    sc = (HD ** -0.5) * 1.4426950408889634

    @pl.when(h == 0)
    def _():
        s = jnp.sum(ab_ref[...]) * sc
        fi = fi_ref[...]
        fj = fj_ref[...]
        fac_ref[...] = (fj * s - fi * s) / (fi * fj + EPS)

    q = q_ref[0] * sc
    k = k_ref[h]
    v = v_ref[h]
    lg = lax.dot_general(q.astype(jnp.bfloat16), k.astype(jnp.bfloat16),
                         (((1,), (1,)), ((), ())),
                         preferred_element_type=jnp.float32)
    lg = lg + fac_ref[...]
    m = jnp.max(lg, axis=1, keepdims=True)
    p = jnp.exp2(lg - m)
    r = jnp.sum(p, axis=1, keepdims=True)
    o = jnp.dot(p, v, preferred_element_type=jnp.float32)
    o_ref[0] = o / r


def _attn(qkv, fr_col, fr_row, attn_bias):
    return pl.pallas_call(
        _attn_body,
        grid=(T // TA, H),
        in_specs=[
            pl.BlockSpec((1, TA, HD), lambda t, h: (h, t, 0)),
            pl.BlockSpec((H, T, HD), lambda t, h: (1, 0, 0)),
            pl.BlockSpec((H, T, HD), lambda t, h: (2, 0, 0)),
            pl.BlockSpec((TA, 1), lambda t, h: (t, 0)),
            pl.BlockSpec((1, T), lambda t, h: (0, 0)),
            pl.BlockSpec((1, HD), lambda t, h: (0, 0)),
        ],
        out_specs=pl.BlockSpec((1, TA, HD), lambda t, h: (h, t, 0)),
        out_shape=jax.ShapeDtypeStruct((H, T, HD), jnp.float32),
        scratch_shapes=[pltpu.VMEM((TA, T), jnp.float32)],
    )(qkv, qkv, qkv, fr_col, fr_row, attn_bias)


def _post_body(ao_ref, wo_ref, bo_ref, src_ref, g1_ref, be1_ref,
               gw_ref, gb_ref, x_ref, gl_ref):
    acc = jnp.dot(ao_ref[...], wo_ref[...], preferred_element_type=jnp.float32)
    x = _ln(src_ref[...] + acc + bo_ref[...], g1_ref[...], be1_ref[...])
    x_ref[...] = x
    gl_ref[...] = jnp.dot(x, gw_ref[...],
                          preferred_element_type=jnp.float32) + gb_ref[...]


def _post_attn(ao, wo, bo, src2, g1, be1, gw, gb):
    return pl.pallas_call(
        _post_body,
        grid=(T // TQ,),
        in_specs=[
            pl.BlockSpec((TQ, D), lambda t: (t, 0)),
            pl.BlockSpec((D, D), lambda t: (0, 0)),
            pl.BlockSpec((1, D), lambda t: (0, 0)),
            pl.BlockSpec((TQ, D), lambda t: (t, 0)),
            pl.BlockSpec((1, D), lambda t: (0, 0)),
            pl.BlockSpec((1, D), lambda t: (0, 0)),
            pl.BlockSpec((D, E), lambda t: (0, 0)),
            pl.BlockSpec((1, E), lambda t: (0, 0)),
        ],
        out_specs=[
            pl.BlockSpec((TQ, D), lambda t: (t, 0)),
            pl.BlockSpec((TQ, E), lambda t: (t, 0)),
        ],
        out_shape=[
            jax.ShapeDtypeStruct((T, D), jnp.float32),
            jax.ShapeDtypeStruct((T, E), jnp.float32),
        ],
    )(ao, wo, bo, src2, g1, be1, gw, gb)


def _route_body(gl_ref, dest_ref, w_ref, bexp_ref, nused_ref):
    gl = gl_ref[...]
    m = jnp.max(gl, axis=1, keepdims=True)
    p = jnp.exp(gl - m)
    g = p / jnp.sum(p, axis=1, keepdims=True)
    ioe = lax.broadcasted_iota(jnp.int32, (T, E), 1)
    m1 = jnp.max(g, axis=1, keepdims=True)
    a1 = jnp.min(jnp.where(g == m1, ioe, E), axis=1, keepdims=True)
    oh1 = (ioe == a1).astype(jnp.float32)
    gm = jnp.where(ioe == a1, -jnp.inf, g)
    m2 = jnp.max(gm, axis=1, keepdims=True)
    a2 = jnp.min(jnp.where(gm == m2, ioe, E), axis=1, keepdims=True)
    oh2 = (ioe == a2).astype(jnp.float32)
    w_ref[...] = jnp.concatenate([m1, m2], axis=1)

    c0 = oh1
    c1 = oh2
    k = 1
    while k < T:
        z = jnp.zeros((k, E), jnp.float32)
        c0 = c0 + jnp.concatenate([z, c0[:-k]], axis=0)
        c1 = c1 + jnp.concatenate([z, c1[:-k]], axis=0)
        k *= 2
    tot = jnp.sum(oh1 + oh2, axis=0, keepdims=True)
    pc = jnp.floor((tot + (BM - 1)) / BM)
    ci = pc
    for k in (1, 2, 4):
        ci = ci + jnp.concatenate(
            [jnp.zeros((1, k), jnp.float32), ci[:, :-k]], axis=1)
    bstart = ci - pc
    sstart = bstart * BM
    nb_used = jnp.sum(pc)

    before0 = (c0 - oh1) + (c1 - oh2)
    before1 = c0 + (c1 - oh2)
    d0 = jnp.sum(oh1 * (sstart + before0), axis=1, keepdims=True)
    d1 = jnp.sum(oh2 * (sstart + before1), axis=1, keepdims=True)
    dest_ref[...] = jnp.concatenate([d0, d1], axis=1).astype(jnp.int32)

    bi = lax.broadcasted_iota(jnp.int32, (NB, E), 0).astype(jnp.float32)
    ind = jnp.logical_and(bi >= bstart, bi < bstart + pc).astype(jnp.float32)
    ev = lax.broadcasted_iota(jnp.int32, (NB, E), 1).astype(jnp.float32)
    eb = jnp.sum(ind * ev, axis=1, keepdims=True)
    last_e = jnp.max(eb)
    bi0 = lax.broadcasted_iota(jnp.int32, (NB, 1), 0).astype(jnp.float32)
    eb = jnp.where(bi0 < nb_used, eb, last_e)
    bexp_ref[...] = eb.astype(jnp.int32)
    nused_ref[...] = jnp.full((1, 1), nb_used, jnp.float32).astype(jnp.int32)


def _route(gl):
    return pl.pallas_call(
        _route_body,
        out_shape=[
            jax.ShapeDtypeStruct((T, 2), jnp.int32),
            jax.ShapeDtypeStruct((T, 2), jnp.float32),
            jax.ShapeDtypeStruct((NB, 1), jnp.int32),
            jax.ShapeDtypeStruct((1, 1), jnp.int32),
        ],
    )(gl)


def _sc_scatter(x, d0, d1):
    mesh = plsc.VectorSubcoreMesh(core_axis_name="c", subcore_axis_name="s")

    @functools.partial(
        pl.kernel, mesh=mesh,
        out_type=jax.ShapeDtypeStruct((NS, D), jnp.float32),
        scratch_types=[
            pltpu.VMEM((CHUNK,), jnp.int32),
            pltpu.VMEM((CHUNK, D), jnp.float32),
            pltpu.SemaphoreType.DMA,
        ],
    )
    def k(x_hbm, d0_hbm, d1_hbm, out_hbm, idx_v, rows_v, sem):
        wid = lax.axis_index("s") * 2 + lax.axis_index("c")
        base = wid * CHUNK
        pltpu.sync_copy(x_hbm.at[pl.ds(base, CHUNK)], rows_v)
        pltpu.sync_copy(d0_hbm.at[pl.ds(base, CHUNK)], idx_v)
        pltpu.async_copy(rows_v, out_hbm.at[idx_v], sem).wait()
        pltpu.sync_copy(d1_hbm.at[pl.ds(base, CHUNK)], idx_v)
        pltpu.async_copy(rows_v, out_hbm.at[idx_v], sem).wait()

    return k(x, d0, d1)


def _ffn_body(be_ref, nu_ref, xs_ref, w1_ref, b1_ref, w2_ref, b2_ref, ys_ref):
    b = pl.program_id(0)

    @pl.when(b < nu_ref[0])
    def _():
        h = jnp.dot(xs_ref[...], w1_ref[0],
                    preferred_element_type=jnp.float32) + b1_ref[0]
        h = jnp.maximum(h, 0.0)
        ys_ref[...] = jnp.dot(h, w2_ref[0],
                              preferred_element_type=jnp.float32) + b2_ref[0]


def _ffn(bexp, nused, xs, w1, b1, w2, b2):
    grid_spec = pltpu.PrefetchScalarGridSpec(
        num_scalar_prefetch=2,
        grid=(NB,),
        in_specs=[
            pl.BlockSpec((BM, D),
                         lambda b, be, nu: (jnp.minimum(b, nu[0] - 1), 0)),
            pl.BlockSpec((1, D, FF), lambda b, be, nu: (be[b], 0, 0)),
            pl.BlockSpec((1, 1, FF), lambda b, be, nu: (be[b], 0, 0)),
            pl.BlockSpec((1, FF, D), lambda b, be, nu: (be[b], 0, 0)),
            pl.BlockSpec((1, 1, D), lambda b, be, nu: (be[b], 0, 0)),
        ],
        out_specs=pl.BlockSpec((BM, D),
                               lambda b, be, nu: (jnp.minimum(b, nu[0] - 1), 0)),
    )
    return pl.pallas_call(
        _ffn_body,
        grid_spec=grid_spec,
        out_shape=jax.ShapeDtypeStruct((NS, D), jnp.float32),
    )(bexp, nused, xs, w1, b1, w2, b2)


def _sc_gather(ys, d0, d1):
    mesh = plsc.VectorSubcoreMesh(core_axis_name="c", subcore_axis_name="s")

    @functools.partial(
        pl.kernel, mesh=mesh,
        out_type=(jax.ShapeDtypeStruct((T, D), jnp.float32),
                  jax.ShapeDtypeStruct((T, D), jnp.float32)),
        scratch_types=[
            pltpu.VMEM((CHUNK,), jnp.int32),
            pltpu.VMEM((CHUNK, D), jnp.float32),
            pltpu.SemaphoreType.DMA,
        ],
    )
    def k(ys_hbm, d0_hbm, d1_hbm, y0_hbm, y1_hbm, idx_v, rows_v, sem):
        wid = lax.axis_index("s") * 2 + lax.axis_index("c")
        base = wid * CHUNK
        pltpu.sync_copy(d0_hbm.at[pl.ds(base, CHUNK)], idx_v)
        pltpu.async_copy(ys_hbm.at[idx_v], rows_v, sem).wait()
        pltpu.sync_copy(rows_v, y0_hbm.at[pl.ds(base, CHUNK)])
        pltpu.sync_copy(d1_hbm.at[pl.ds(base, CHUNK)], idx_v)
        pltpu.async_copy(ys_hbm.at[idx_v], rows_v, sem).wait()
        pltpu.sync_copy(rows_v, y1_hbm.at[pl.ds(base, CHUNK)])

    return k(ys, d0, d1)


def _comb_body(x_ref, y0_ref, y1_ref, w_ref, g2_ref, be2_ref, o_ref):
    w = w_ref[...]
    ff = w[:, 0:1] * y0_ref[...] + w[:, 1:2] * y1_ref[...]
    o_ref[...] = _ln(x_ref[...] + ff, g2_ref[...], be2_ref[...])


def _combine(x, y0, y1, w, g2, be2):
    return pl.pallas_call(
        _comb_body,
        grid=(T // TQ,),
        in_specs=[
            pl.BlockSpec((TQ, D), lambda t: (t, 0)),
            pl.BlockSpec((TQ, D), lambda t: (t, 0)),
            pl.BlockSpec((TQ, D), lambda t: (t, 0)),
            pl.BlockSpec((TQ, 2), lambda t: (t, 0)),
            pl.BlockSpec((1, D), lambda t: (0, 0)),
            pl.BlockSpec((1, D), lambda t: (0, 0)),
        ],
        out_specs=pl.BlockSpec((TQ, D), lambda t: (t, 0)),
        out_shape=jax.ShapeDtypeStruct((T, D), jnp.float32),
    )(x, y0, y1, w, g2, be2)


def kernel(src, frac, Wq, bq, Wk, bk, Wv, bv, attn_bias, Wo, bo,
           gate_w, gate_b, W1, b1, W2, b2, g1, be1, g2, be2):
    src2 = src[0]
    fr = frac[0]
    w3 = jnp.concatenate([Wq, Wk, Wv], axis=1)
    b3 = jnp.concatenate([bq, bk, bv]).reshape(1, 3 * D)

    qkv = _qkv(src2, w3, b3)
    ao = _attn(qkv, fr.reshape(T, 1), fr.reshape(1, T),
               attn_bias.reshape(1, HD))
    aot = ao.transpose(1, 0, 2).reshape(T, D)
    x, gl = _post_attn(aot, Wo, bo.reshape(1, D), src2,
                       g1.reshape(1, D), be1.reshape(1, D),
                       gate_w, gate_b.reshape(1, E))
    dest, w2s, bexp, nused = _route(gl)
    d0 = dest[:, 0]
    d1 = dest[:, 1]
    xs = _sc_scatter(x, d0, d1)
    ys = _ffn(bexp.reshape(NB), nused.reshape(1), xs,
              W1, b1.reshape(E, 1, FF), W2, b2.reshape(E, 1, D))
    y0, y1 = _sc_gather(ys, d0, d1)
    y = _combine(x, y0, y1, w2s, g2.reshape(1, D), be2.reshape(1, D))
    return y.reshape(1, T, D)


# speedup vs baseline: 1.0321x; 1.0321x over previous
import functools

import jax
import jax.numpy as jnp
from jax import lax
from jax.experimental import pallas as pl
from jax.experimental.pallas import tpu as pltpu
from jax.experimental.pallas import tpu_sc as plsc

D = 768
H = 12
HD = 64
FF = 2048
E = 8
T = 2048
EPS = 1e-8
LN_EPS = 1e-5

TQ = 256
TA = 256
BM = 256
NS = T * 2 + E * BM
NB = NS // BM
NW = 32
CHUNK = T // NW


def _ln(x, g, b):
    m = jnp.mean(x, axis=-1, keepdims=True)
    v = jnp.mean((x - m) ** 2, axis=-1, keepdims=True)
    return (x - m) * jax.lax.rsqrt(v + LN_EPS) * g + b


def _qkv_body(s_ref, w_ref, b_ref, o_ref):
    o = jnp.dot(s_ref[...], w_ref[...],
                preferred_element_type=jnp.float32) + b_ref[...]
    o_ref[...] = o.reshape(T, 4, HD).transpose(1, 0, 2)


def _qkv(src2, w3, b3):
    BN = 256
    return pl.pallas_call(
        _qkv_body,
        grid=(3 * D // BN,),
        in_specs=[
            pl.BlockSpec((T, D), lambda n: (0, 0)),
            pl.BlockSpec((D, BN), lambda n: (0, n)),
            pl.BlockSpec((1, BN), lambda n: (0, n)),
        ],
        out_specs=pl.BlockSpec((4, T, HD), lambda n: (n, 0, 0)),
        out_shape=jax.ShapeDtypeStruct((3 * H, T, HD), jnp.float32),
    )(src2, w3, b3)


def _attn_body(q_ref, k_ref, v_ref, fi_ref, fj_ref, ab_ref, o_ref, fac_ref):
    h = pl.program_id(1)
    sc = (HD ** -0.5) * 1.4426950408889634

    @pl.when(h == 0)
    def _():
        s = jnp.sum(ab_ref[...]) * sc
        fi = fi_ref[...]
        fj = fj_ref[...]
        fac_ref[...] = (fj * s - fi * s) / (fi * fj + EPS)

    q = q_ref[0] * sc
    k = k_ref[h]
    v = v_ref[h]
    lg = lax.dot_general(q.astype(jnp.bfloat16), k.astype(jnp.bfloat16),
                         (((1,), (1,)), ((), ())),
                         preferred_element_type=jnp.float32)
    lg = lg + fac_ref[...]
    m = jnp.max(lg, axis=1, keepdims=True)
    p = jnp.exp2(lg - m)
    r = jnp.sum(p, axis=1, keepdims=True)
    pb = p.astype(jnp.bfloat16)
    v_hi = v.astype(jnp.bfloat16)
    v_lo = (v - v_hi.astype(jnp.float32)).astype(jnp.bfloat16)
    o = (jnp.dot(pb, v_hi, preferred_element_type=jnp.float32)
         + jnp.dot(pb, v_lo, preferred_element_type=jnp.float32))
    o_ref[0] = o / r


def _attn(qkv, fr_col, fr_row, attn_bias):
    return pl.pallas_call(
        _attn_body,
        grid=(T // TA, H),
        in_specs=[
            pl.BlockSpec((1, TA, HD), lambda t, h: (h, t, 0)),
            pl.BlockSpec((H, T, HD), lambda t, h: (1, 0, 0)),
            pl.BlockSpec((H, T, HD), lambda t, h: (2, 0, 0)),
            pl.BlockSpec((TA, 1), lambda t, h: (t, 0)),
            pl.BlockSpec((1, T), lambda t, h: (0, 0)),
            pl.BlockSpec((1, HD), lambda t, h: (0, 0)),
        ],
        out_specs=pl.BlockSpec((1, TA, HD), lambda t, h: (h, t, 0)),
        out_shape=jax.ShapeDtypeStruct((H, T, HD), jnp.float32),
        scratch_shapes=[pltpu.VMEM((TA, T), jnp.float32)],
    )(qkv, qkv, qkv, fr_col, fr_row, attn_bias)


def _post_body(ao_ref, wo_ref, bo_ref, src_ref, g1_ref, be1_ref,
               gw_ref, gb_ref, x_ref, gl_ref):
    acc = jnp.dot(ao_ref[...], wo_ref[...], preferred_element_type=jnp.float32)
    x = _ln(src_ref[...] + acc + bo_ref[...], g1_ref[...], be1_ref[...])
    x_ref[...] = x
    gl_ref[...] = jnp.dot(x, gw_ref[...],
                          preferred_element_type=jnp.float32) + gb_ref[...]


def _post_attn(ao, wo, bo, src2, g1, be1, gw, gb):
    return pl.pallas_call(
        _post_body,
        grid=(T // TQ,),
        in_specs=[
            pl.BlockSpec((TQ, D), lambda t: (t, 0)),
            pl.BlockSpec((D, D), lambda t: (0, 0)),
            pl.BlockSpec((1, D), lambda t: (0, 0)),
            pl.BlockSpec((TQ, D), lambda t: (t, 0)),
            pl.BlockSpec((1, D), lambda t: (0, 0)),
            pl.BlockSpec((1, D), lambda t: (0, 0)),
            pl.BlockSpec((D, E), lambda t: (0, 0)),
            pl.BlockSpec((1, E), lambda t: (0, 0)),
        ],
        out_specs=[
            pl.BlockSpec((TQ, D), lambda t: (t, 0)),
            pl.BlockSpec((TQ, E), lambda t: (t, 0)),
        ],
        out_shape=[
            jax.ShapeDtypeStruct((T, D), jnp.float32),
            jax.ShapeDtypeStruct((T, E), jnp.float32),
        ],
    )(ao, wo, bo, src2, g1, be1, gw, gb)


def _route_body(gl_ref, d0_ref, d1_ref, w_ref, bexp_ref, nused_ref):
    gl = gl_ref[...]
    m = jnp.max(gl, axis=1, keepdims=True)
    p = jnp.exp(gl - m)
    g = p / jnp.sum(p, axis=1, keepdims=True)
    ioe = lax.broadcasted_iota(jnp.int32, (T, E), 1)
    m1 = jnp.max(g, axis=1, keepdims=True)
    a1 = jnp.min(jnp.where(g == m1, ioe, E), axis=1, keepdims=True)
    oh1 = (ioe == a1).astype(jnp.float32)
    gm = jnp.where(ioe == a1, -jnp.inf, g)
    m2 = jnp.max(gm, axis=1, keepdims=True)
    a2 = jnp.min(jnp.where(gm == m2, ioe, E), axis=1, keepdims=True)
    oh2 = (ioe == a2).astype(jnp.float32)
    w_ref[...] = jnp.concatenate([m1, m2], axis=1)

    c0 = oh1
    c1 = oh2
    k = 1
    while k < T:
        z = jnp.zeros((k, E), jnp.float32)
        c0 = c0 + jnp.concatenate([z, c0[:-k]], axis=0)
        c1 = c1 + jnp.concatenate([z, c1[:-k]], axis=0)
        k *= 2
    tot = jnp.sum(oh1 + oh2, axis=0, keepdims=True)
    pc = jnp.floor((tot + (BM - 1)) / BM)
    ci = pc
    for k in (1, 2, 4):
        ci = ci + jnp.concatenate(
            [jnp.zeros((1, k), jnp.float32), ci[:, :-k]], axis=1)
    bstart = ci - pc
    sstart = bstart * BM
    nb_used = jnp.sum(pc)

    before0 = (c0 - oh1) + (c1 - oh2)
    before1 = c0 + (c1 - oh2)
    d0 = jnp.sum(oh1 * (sstart + before0), axis=1, keepdims=True)
    d1 = jnp.sum(oh2 * (sstart + before1), axis=1, keepdims=True)
    d0_ref[...] = d0.astype(jnp.int32)
    d1_ref[...] = d1.astype(jnp.int32)

    bi = lax.broadcasted_iota(jnp.int32, (NB, E), 0).astype(jnp.float32)
    ind = jnp.logical_and(bi >= bstart, bi < bstart + pc).astype(jnp.float32)
    ev = lax.broadcasted_iota(jnp.int32, (NB, E), 1).astype(jnp.float32)
    eb = jnp.sum(ind * ev, axis=1, keepdims=True)
    last_e = jnp.max(eb)
    bi0 = lax.broadcasted_iota(jnp.int32, (NB, 1), 0).astype(jnp.float32)
    eb = jnp.where(bi0 < nb_used, eb, last_e)
    bexp_ref[...] = eb.astype(jnp.int32)
    nused_ref[...] = jnp.full((1, 1), nb_used, jnp.float32).astype(jnp.int32)


def _route(gl):
    return pl.pallas_call(
        _route_body,
        out_shape=[
            jax.ShapeDtypeStruct((T, 1), jnp.int32),
            jax.ShapeDtypeStruct((T, 1), jnp.int32),
            jax.ShapeDtypeStruct((T, 2), jnp.float32),
            jax.ShapeDtypeStruct((NB, 1), jnp.int32),
            jax.ShapeDtypeStruct((1, 1), jnp.int32),
        ],
    )(gl)


def _sc_scatter(x, d0, d1):
    mesh = plsc.VectorSubcoreMesh(core_axis_name="c", subcore_axis_name="s")

    @functools.partial(
        pl.kernel, mesh=mesh,
        out_type=jax.ShapeDtypeStruct((NS, D), jnp.float32),
        scratch_types=[
            pltpu.VMEM((CHUNK,), jnp.int32),
            pltpu.VMEM((CHUNK, D), jnp.float32),
            pltpu.SemaphoreType.DMA,
        ],
    )
    def k(x_hbm, d0_hbm, d1_hbm, out_hbm, idx_v, rows_v, sem):
        wid = lax.axis_index("s") * 2 + lax.axis_index("c")
        base = wid * CHUNK
        pltpu.sync_copy(x_hbm.at[pl.ds(base, CHUNK)], rows_v)
        pltpu.sync_copy(d0_hbm.at[pl.ds(base, CHUNK)], idx_v)
        pltpu.async_copy(rows_v, out_hbm.at[idx_v], sem).wait()
        pltpu.sync_copy(d1_hbm.at[pl.ds(base, CHUNK)], idx_v)
        pltpu.async_copy(rows_v, out_hbm.at[idx_v], sem).wait()

    return k(x, d0, d1)


def _ffn_body(be_ref, nu_ref, xs_ref, w1_ref, b1_ref, w2_ref, b2_ref, ys_ref):
    b = pl.program_id(0)

    @pl.when(b < nu_ref[0])
    def _():
        h = jnp.dot(xs_ref[...], w1_ref[0],
                    preferred_element_type=jnp.float32) + b1_ref[0]
        h = jnp.maximum(h, 0.0)
        ys_ref[...] = jnp.dot(h, w2_ref[0],
                              preferred_element_type=jnp.float32) + b2_ref[0]


def _ffn(bexp, nused, xs, w1, b1, w2, b2):
    grid_spec = pltpu.PrefetchScalarGridSpec(
        num_scalar_prefetch=2,
        grid=(NB,),
        in_specs=[
            pl.BlockSpec((BM, D),
                         lambda b, be, nu: (jnp.minimum(b, nu[0] - 1), 0)),
            pl.BlockSpec((1, D, FF), lambda b, be, nu: (be[b], 0, 0)),
            pl.BlockSpec((1, 1, FF), lambda b, be, nu: (be[b], 0, 0)),
            pl.BlockSpec((1, FF, D), lambda b, be, nu: (be[b], 0, 0)),
            pl.BlockSpec((1, 1, D), lambda b, be, nu: (be[b], 0, 0)),
        ],
        out_specs=pl.BlockSpec((BM, D),
                               lambda b, be, nu: (jnp.minimum(b, nu[0] - 1), 0)),
    )
    return pl.pallas_call(
        _ffn_body,
        grid_spec=grid_spec,
        out_shape=jax.ShapeDtypeStruct((NS, D), jnp.float32),
    )(bexp, nused, xs, w1, b1, w2, b2)


def _sc_gather(ys, d0, d1):
    mesh = plsc.VectorSubcoreMesh(core_axis_name="c", subcore_axis_name="s")

    @functools.partial(
        pl.kernel, mesh=mesh,
        out_type=(jax.ShapeDtypeStruct((T, D), jnp.float32),
                  jax.ShapeDtypeStruct((T, D), jnp.float32)),
        scratch_types=[
            pltpu.VMEM((CHUNK,), jnp.int32),
            pltpu.VMEM((CHUNK, D), jnp.float32),
            pltpu.SemaphoreType.DMA,
        ],
    )
    def k(ys_hbm, d0_hbm, d1_hbm, y0_hbm, y1_hbm, idx_v, rows_v, sem):
        wid = lax.axis_index("s") * 2 + lax.axis_index("c")
        base = wid * CHUNK
        pltpu.sync_copy(d0_hbm.at[pl.ds(base, CHUNK)], idx_v)
        pltpu.async_copy(ys_hbm.at[idx_v], rows_v, sem).wait()
        pltpu.sync_copy(rows_v, y0_hbm.at[pl.ds(base, CHUNK)])
        pltpu.sync_copy(d1_hbm.at[pl.ds(base, CHUNK)], idx_v)
        pltpu.async_copy(ys_hbm.at[idx_v], rows_v, sem).wait()
        pltpu.sync_copy(rows_v, y1_hbm.at[pl.ds(base, CHUNK)])

    return k(ys, d0, d1)


def _comb_body(x_ref, y0_ref, y1_ref, w_ref, g2_ref, be2_ref, o_ref):
    w = w_ref[...]
    ff = w[:, 0:1] * y0_ref[...] + w[:, 1:2] * y1_ref[...]
    o_ref[...] = _ln(x_ref[...] + ff, g2_ref[...], be2_ref[...])


def _combine(x, y0, y1, w, g2, be2):
    return pl.pallas_call(
        _comb_body,
        grid=(T // TQ,),
        in_specs=[
            pl.BlockSpec((TQ, D), lambda t: (t, 0)),
            pl.BlockSpec((TQ, D), lambda t: (t, 0)),
            pl.BlockSpec((TQ, D), lambda t: (t, 0)),
            pl.BlockSpec((TQ, 2), lambda t: (t, 0)),
            pl.BlockSpec((1, D), lambda t: (0, 0)),
            pl.BlockSpec((1, D), lambda t: (0, 0)),
        ],
        out_specs=pl.BlockSpec((TQ, D), lambda t: (t, 0)),
        out_shape=jax.ShapeDtypeStruct((T, D), jnp.float32),
    )(x, y0, y1, w, g2, be2)


def kernel(src, frac, Wq, bq, Wk, bk, Wv, bv, attn_bias, Wo, bo,
           gate_w, gate_b, W1, b1, W2, b2, g1, be1, g2, be2):
    src2 = src[0]
    fr = frac[0]
    w3 = jnp.concatenate([Wq, Wk, Wv], axis=1)
    b3 = jnp.concatenate([bq, bk, bv]).reshape(1, 3 * D)

    qkv = _qkv(src2, w3, b3)
    ao = _attn(qkv, fr.reshape(T, 1), fr.reshape(1, T),
               attn_bias.reshape(1, HD))
    aot = ao.transpose(1, 0, 2).reshape(T, D)
    x, gl = _post_attn(aot, Wo, bo.reshape(1, D), src2,
                       g1.reshape(1, D), be1.reshape(1, D),
                       gate_w, gate_b.reshape(1, E))
    d0c, d1c, w2s, bexp, nused = _route(gl)
    d0 = d0c.reshape(T)
    d1 = d1c.reshape(T)
    xs = _sc_scatter(x, d0, d1)
    ys = _ffn(bexp.reshape(NB), nused.reshape(1), xs,
              W1, b1.reshape(E, 1, FF), W2, b2.reshape(E, 1, D))
    y0, y1 = _sc_gather(ys, d0, d1)
    y = _combine(x, y0, y1, w2s, g2.reshape(1, D), be2.reshape(1, D))
    return y.reshape(1, T, D)


# speedup vs baseline: 1.1197x; 1.0849x over previous
import functools

import jax
import jax.numpy as jnp
from jax import lax
from jax.experimental import pallas as pl
from jax.experimental.pallas import tpu as pltpu
from jax.experimental.pallas import tpu_sc as plsc

D = 768
H = 12
HD = 64
FF = 2048
E = 8
T = 2048
EPS = 1e-8
LN_EPS = 1e-5

TQ = 256
TA = 256
BM = 256
NS = T * 2 + E * BM
NB = NS // BM
NW = 32
CHUNK = T // NW


def _ln(x, g, b):
    m = jnp.mean(x, axis=-1, keepdims=True)
    v = jnp.mean((x - m) ** 2, axis=-1, keepdims=True)
    return (x - m) * jax.lax.rsqrt(v + LN_EPS) * g + b


def _qkv_body(s_ref, w_ref, b_ref, o_ref):
    o = jnp.dot(s_ref[...], w_ref[...],
                preferred_element_type=jnp.float32) + b_ref[...]
    o_ref[...] = o.reshape(T, 4, HD).transpose(1, 0, 2)


def _qkv(src2, w3, b3):
    BN = 256
    return pl.pallas_call(
        _qkv_body,
        grid=(3 * D // BN,),
        in_specs=[
            pl.BlockSpec((T, D), lambda n: (0, 0)),
            pl.BlockSpec((D, BN), lambda n: (0, n)),
            pl.BlockSpec((1, BN), lambda n: (0, n)),
        ],
        out_specs=pl.BlockSpec((4, T, HD), lambda n: (n, 0, 0)),
        out_shape=jax.ShapeDtypeStruct((3 * H, T, HD), jnp.float32),
    )(src2, w3, b3)


def _attn_body(q_ref, k_ref, v_ref, fi_ref, fj_ref, ab_ref, o_ref, fac_ref):
    h = pl.program_id(1)
    sc = (HD ** -0.5) * 1.4426950408889634

    @pl.when(h == 0)
    def _():
        s = jnp.sum(ab_ref[...]) * sc
        fi = fi_ref[...]
        fj = fj_ref[...]
        fac_ref[...] = (fj * s - fi * s) / (fi * fj + EPS)

    q = q_ref[0] * sc
    k = k_ref[h]
    v = v_ref[h]
    lg = lax.dot_general(q.astype(jnp.bfloat16), k.astype(jnp.bfloat16),
                         (((1,), (1,)), ((), ())),
                         preferred_element_type=jnp.float32)
    lg = lg + fac_ref[...]
    m = jnp.max(lg, axis=1, keepdims=True)
    p = jnp.exp2(lg - m)
    r = jnp.sum(p, axis=1, keepdims=True)
    o = jnp.dot(p.astype(jnp.bfloat16), v.astype(jnp.bfloat16),
                preferred_element_type=jnp.float32)
    o_ref[0] = o / r


def _attn(qkv, fr_col, fr_row, attn_bias):
    return pl.pallas_call(
        _attn_body,
        grid=(T // TA, H),
        in_specs=[
            pl.BlockSpec((1, TA, HD), lambda t, h: (h, t, 0)),
            pl.BlockSpec((H, T, HD), lambda t, h: (1, 0, 0)),
            pl.BlockSpec((H, T, HD), lambda t, h: (2, 0, 0)),
            pl.BlockSpec((TA, 1), lambda t, h: (t, 0)),
            pl.BlockSpec((1, T), lambda t, h: (0, 0)),
            pl.BlockSpec((1, HD), lambda t, h: (0, 0)),
        ],
        out_specs=pl.BlockSpec((1, TA, HD), lambda t, h: (h, t, 0)),
        out_shape=jax.ShapeDtypeStruct((H, T, HD), jnp.float32),
        scratch_shapes=[pltpu.VMEM((TA, T), jnp.float32)],
    )(qkv, qkv, qkv, fr_col, fr_row, attn_bias)


def _post_body(ao_ref, wo_ref, bo_ref, src_ref, g1_ref, be1_ref,
               gw_ref, gb_ref, x_ref, gl_ref):
    acc = jnp.dot(ao_ref[...], wo_ref[...], preferred_element_type=jnp.float32)
    x = _ln(src_ref[...] + acc + bo_ref[...], g1_ref[...], be1_ref[...])
    x_ref[...] = x
    gl_ref[...] = jnp.dot(x, gw_ref[...],
                          preferred_element_type=jnp.float32) + gb_ref[...]


def _post_attn(ao, wo, bo, src2, g1, be1, gw, gb):
    return pl.pallas_call(
        _post_body,
        grid=(T // TQ,),
        in_specs=[
            pl.BlockSpec((TQ, D), lambda t: (t, 0)),
            pl.BlockSpec((D, D), lambda t: (0, 0)),
            pl.BlockSpec((1, D), lambda t: (0, 0)),
            pl.BlockSpec((TQ, D), lambda t: (t, 0)),
            pl.BlockSpec((1, D), lambda t: (0, 0)),
            pl.BlockSpec((1, D), lambda t: (0, 0)),
            pl.BlockSpec((D, E), lambda t: (0, 0)),
            pl.BlockSpec((1, E), lambda t: (0, 0)),
        ],
        out_specs=[
            pl.BlockSpec((TQ, D), lambda t: (t, 0)),
            pl.BlockSpec((TQ, E), lambda t: (t, 0)),
        ],
        out_shape=[
            jax.ShapeDtypeStruct((T, D), jnp.float32),
            jax.ShapeDtypeStruct((T, E), jnp.float32),
        ],
    )(ao, wo, bo, src2, g1, be1, gw, gb)


def _route_body(gl_ref, d0_ref, d1_ref, w_ref, bexp_ref, nused_ref):
    gl = gl_ref[...]
    m = jnp.max(gl, axis=1, keepdims=True)
    p = jnp.exp(gl - m)
    g = p / jnp.sum(p, axis=1, keepdims=True)
    ioe = lax.broadcasted_iota(jnp.int32, (T, E), 1)
    m1 = jnp.max(g, axis=1, keepdims=True)
    a1 = jnp.min(jnp.where(g == m1, ioe, E), axis=1, keepdims=True)
    oh1 = (ioe == a1).astype(jnp.float32)
    gm = jnp.where(ioe == a1, -jnp.inf, g)
    m2 = jnp.max(gm, axis=1, keepdims=True)
    a2 = jnp.min(jnp.where(gm == m2, ioe, E), axis=1, keepdims=True)
    oh2 = (ioe == a2).astype(jnp.float32)
    w_ref[...] = jnp.concatenate([m1, m2], axis=1)

    c0 = oh1
    c1 = oh2
    k = 1
    while k < T:
        z = jnp.zeros((k, E), jnp.float32)
        c0 = c0 + jnp.concatenate([z, c0[:-k]], axis=0)
        c1 = c1 + jnp.concatenate([z, c1[:-k]], axis=0)
        k *= 2
    tot = jnp.sum(oh1 + oh2, axis=0, keepdims=True)
    pc = jnp.floor((tot + (BM - 1)) / BM)
    ci = pc
    for k in (1, 2, 4):
        ci = ci + jnp.concatenate(
            [jnp.zeros((1, k), jnp.float32), ci[:, :-k]], axis=1)
    bstart = ci - pc
    sstart = bstart * BM
    nb_used = jnp.sum(pc)

    before0 = (c0 - oh1) + (c1 - oh2)
    before1 = c0 + (c1 - oh2)
    d0 = jnp.sum(oh1 * (sstart + before0), axis=1, keepdims=True)
    d1 = jnp.sum(oh2 * (sstart + before1), axis=1, keepdims=True)
    d0_ref[...] = d0.astype(jnp.int32)
    d1_ref[...] = d1.astype(jnp.int32)

    bi = lax.broadcasted_iota(jnp.int32, (NB, E), 0).astype(jnp.float32)
    ind = jnp.logical_and(bi >= bstart, bi < bstart + pc).astype(jnp.float32)
    ev = lax.broadcasted_iota(jnp.int32, (NB, E), 1).astype(jnp.float32)
    eb = jnp.sum(ind * ev, axis=1, keepdims=True)
    last_e = jnp.max(eb)
    bi0 = lax.broadcasted_iota(jnp.int32, (NB, 1), 0).astype(jnp.float32)
    eb = jnp.where(bi0 < nb_used, eb, last_e)
    bexp_ref[...] = eb.astype(jnp.int32)
    nused_ref[...] = jnp.full((1, 1), nb_used, jnp.float32).astype(jnp.int32)


def _route(gl):
    return pl.pallas_call(
        _route_body,
        out_shape=[
            jax.ShapeDtypeStruct((T, 1), jnp.int32),
            jax.ShapeDtypeStruct((T, 1), jnp.int32),
            jax.ShapeDtypeStruct((T, 2), jnp.float32),
            jax.ShapeDtypeStruct((NB, 1), jnp.int32),
            jax.ShapeDtypeStruct((1, 1), jnp.int32),
        ],
    )(gl)


def _sc_scatter(x, d0, d1):
    mesh = plsc.VectorSubcoreMesh(core_axis_name="c", subcore_axis_name="s")

    @functools.partial(
        pl.kernel, mesh=mesh,
        out_type=jax.ShapeDtypeStruct((NS, D), jnp.float32),
        scratch_types=[
            pltpu.VMEM((CHUNK,), jnp.int32),
            pltpu.VMEM((CHUNK, D), jnp.float32),
            pltpu.SemaphoreType.DMA,
        ],
    )
    def k(x_hbm, d0_hbm, d1_hbm, out_hbm, idx_v, rows_v, sem):
        wid = lax.axis_index("s") * 2 + lax.axis_index("c")
        base = wid * CHUNK
        pltpu.sync_copy(x_hbm.at[pl.ds(base, CHUNK)], rows_v)
        pltpu.sync_copy(d0_hbm.at[pl.ds(base, CHUNK)], idx_v)
        pltpu.async_copy(rows_v, out_hbm.at[idx_v], sem).wait()
        pltpu.sync_copy(d1_hbm.at[pl.ds(base, CHUNK)], idx_v)
        pltpu.async_copy(rows_v, out_hbm.at[idx_v], sem).wait()

    return k(x, d0, d1)


def _ffn_body(be_ref, nu_ref, xs_ref, w1_ref, b1_ref, w2_ref, b2_ref, ys_ref):
    b = pl.program_id(0)

    @pl.when(b < nu_ref[0])
    def _():
        h = jnp.dot(xs_ref[...], w1_ref[0],
                    preferred_element_type=jnp.float32) + b1_ref[0]
        h = jnp.maximum(h, 0.0)
        ys_ref[...] = jnp.dot(h, w2_ref[0],
                              preferred_element_type=jnp.float32) + b2_ref[0]


def _ffn(bexp, nused, xs, w1, b1, w2, b2):
    grid_spec = pltpu.PrefetchScalarGridSpec(
        num_scalar_prefetch=2,
        grid=(NB,),
        in_specs=[
            pl.BlockSpec((BM, D),
                         lambda b, be, nu: (jnp.minimum(b, nu[0] - 1), 0)),
            pl.BlockSpec((1, D, FF), lambda b, be, nu: (be[b], 0, 0)),
            pl.BlockSpec((1, 1, FF), lambda b, be, nu: (be[b], 0, 0)),
            pl.BlockSpec((1, FF, D), lambda b, be, nu: (be[b], 0, 0)),
            pl.BlockSpec((1, 1, D), lambda b, be, nu: (be[b], 0, 0)),
        ],
        out_specs=pl.BlockSpec((BM, D),
                               lambda b, be, nu: (jnp.minimum(b, nu[0] - 1), 0)),
    )
    return pl.pallas_call(
        _ffn_body,
        grid_spec=grid_spec,
        out_shape=jax.ShapeDtypeStruct((NS, D), jnp.float32),
    )(bexp, nused, xs, w1, b1, w2, b2)


def _sc_gather(ys, d0, d1):
    mesh = plsc.VectorSubcoreMesh(core_axis_name="c", subcore_axis_name="s")

    @functools.partial(
        pl.kernel, mesh=mesh,
        out_type=(jax.ShapeDtypeStruct((T, D), jnp.float32),
                  jax.ShapeDtypeStruct((T, D), jnp.float32)),
        scratch_types=[
            pltpu.VMEM((CHUNK,), jnp.int32),
            pltpu.VMEM((CHUNK, D), jnp.float32),
            pltpu.SemaphoreType.DMA,
        ],
    )
    def k(ys_hbm, d0_hbm, d1_hbm, y0_hbm, y1_hbm, idx_v, rows_v, sem):
        wid = lax.axis_index("s") * 2 + lax.axis_index("c")
        base = wid * CHUNK
        pltpu.sync_copy(d0_hbm.at[pl.ds(base, CHUNK)], idx_v)
        pltpu.async_copy(ys_hbm.at[idx_v], rows_v, sem).wait()
        pltpu.sync_copy(rows_v, y0_hbm.at[pl.ds(base, CHUNK)])
        pltpu.sync_copy(d1_hbm.at[pl.ds(base, CHUNK)], idx_v)
        pltpu.async_copy(ys_hbm.at[idx_v], rows_v, sem).wait()
        pltpu.sync_copy(rows_v, y1_hbm.at[pl.ds(base, CHUNK)])

    return k(ys, d0, d1)


def _comb_body(x_ref, y0_ref, y1_ref, w_ref, g2_ref, be2_ref, o_ref):
    w = w_ref[...]
    ff = w[:, 0:1] * y0_ref[...] + w[:, 1:2] * y1_ref[...]
    o_ref[...] = _ln(x_ref[...] + ff, g2_ref[...], be2_ref[...])


def _combine(x, y0, y1, w, g2, be2):
    return pl.pallas_call(
        _comb_body,
        grid=(T // TQ,),
        in_specs=[
            pl.BlockSpec((TQ, D), lambda t: (t, 0)),
            pl.BlockSpec((TQ, D), lambda t: (t, 0)),
            pl.BlockSpec((TQ, D), lambda t: (t, 0)),
            pl.BlockSpec((TQ, 2), lambda t: (t, 0)),
            pl.BlockSpec((1, D), lambda t: (0, 0)),
            pl.BlockSpec((1, D), lambda t: (0, 0)),
        ],
        out_specs=pl.BlockSpec((TQ, D), lambda t: (t, 0)),
        out_shape=jax.ShapeDtypeStruct((T, D), jnp.float32),
    )(x, y0, y1, w, g2, be2)


def kernel(src, frac, Wq, bq, Wk, bk, Wv, bv, attn_bias, Wo, bo,
           gate_w, gate_b, W1, b1, W2, b2, g1, be1, g2, be2):
    src2 = src[0]
    fr = frac[0]
    w3 = jnp.concatenate([Wq, Wk, Wv], axis=1)
    b3 = jnp.concatenate([bq, bk, bv]).reshape(1, 3 * D)

    qkv = _qkv(src2, w3, b3)
    ao = _attn(qkv, fr.reshape(T, 1), fr.reshape(1, T),
               attn_bias.reshape(1, HD))
    aot = ao.transpose(1, 0, 2).reshape(T, D)
    x, gl = _post_attn(aot, Wo, bo.reshape(1, D), src2,
                       g1.reshape(1, D), be1.reshape(1, D),
                       gate_w, gate_b.reshape(1, E))
    d0c, d1c, w2s, bexp, nused = _route(gl)
    d0 = d0c.reshape(T)
    d1 = d1c.reshape(T)
    xs = _sc_scatter(x, d0, d1)
    ys = _ffn(bexp.reshape(NB), nused.reshape(1), xs,
              W1, b1.reshape(E, 1, FF), W2, b2.reshape(E, 1, D))
    y0, y1 = _sc_gather(ys, d0, d1)
    y = _combine(x, y0, y1, w2s, g2.reshape(1, D), be2.reshape(1, D))
    return y.reshape(1, T, D)


# speedup vs baseline: 1.1413x; 1.0193x over previous
import functools

import jax
import jax.numpy as jnp
from jax import lax
from jax.experimental import pallas as pl
from jax.experimental.pallas import tpu as pltpu
from jax.experimental.pallas import tpu_sc as plsc

D = 768
H = 12
HD = 64
FF = 2048
E = 8
T = 2048
EPS = 1e-8
LN_EPS = 1e-5

TQ = 256
TA = 256
BM = 256
NS = T * 2 + E * BM
NB = NS // BM
NW = 32
CHUNK = T // NW


def _ln(x, g, b):
    m = jnp.mean(x, axis=-1, keepdims=True)
    v = jnp.mean((x - m) ** 2, axis=-1, keepdims=True)
    return (x - m) * jax.lax.rsqrt(v + LN_EPS) * g + b


def _qkv_body(s_ref, w_ref, b_ref, o_ref):
    o = jnp.dot(s_ref[...], w_ref[...],
                preferred_element_type=jnp.float32) + b_ref[...]
    o_ref[...] = o.reshape(T, 4, HD).transpose(1, 0, 2).astype(jnp.bfloat16)


def _qkv(src2, w3, b3):
    BN = 256
    return pl.pallas_call(
        _qkv_body,
        grid=(3 * D // BN,),
        in_specs=[
            pl.BlockSpec((T, D), lambda n: (0, 0)),
            pl.BlockSpec((D, BN), lambda n: (0, n)),
            pl.BlockSpec((1, BN), lambda n: (0, n)),
        ],
        out_specs=pl.BlockSpec((4, T, HD), lambda n: (n, 0, 0)),
        out_shape=jax.ShapeDtypeStruct((3 * H, T, HD), jnp.bfloat16),
    )(src2, w3, b3)


def _attn_body(q_ref, k_ref, v_ref, fi_ref, fj_ref, ab_ref, o_ref,
               fac_ref, vx_ref):
    t = pl.program_id(0)
    h = pl.program_id(1)
    csc = (HD ** -0.5) * 1.4426950408889634

    @pl.when(h == 0)
    def _():
        s = jnp.sum(ab_ref[...]) * csc
        fi = fi_ref[...]
        fj = fj_ref[...]
        fac_ref[...] = (fj * s - fi * s) / (fi * fj + EPS)

    @pl.when(t == 0)
    def _():
        vx_ref[h, :, 0:HD] = v_ref[h]
        vx_ref[h, :, HD:2 * HD] = jnp.ones((T, HD), jnp.bfloat16)

    q = q_ref[0]
    k = k_ref[h]
    lg = lax.dot_general(q, k, (((1,), (1,)), ((), ())),
                         preferred_element_type=jnp.float32)
    lg = lg + fac_ref[...]
    m = jnp.max(lg, axis=1, keepdims=True)
    p = jnp.exp2(lg - m)
    ox = jnp.dot(p.astype(jnp.bfloat16), vx_ref[h],
                 preferred_element_type=jnp.float32)
    o_ref[0] = ox[:, 0:HD] / ox[:, HD:HD + 1]


def _attn(qkv, fr_col, fr_row, attn_bias):
    return pl.pallas_call(
        _attn_body,
        grid=(T // TA, H),
        in_specs=[
            pl.BlockSpec((1, TA, HD), lambda t, h: (h, t, 0)),
            pl.BlockSpec((H, T, HD), lambda t, h: (1, 0, 0)),
            pl.BlockSpec((H, T, HD), lambda t, h: (2, 0, 0)),
            pl.BlockSpec((TA, 1), lambda t, h: (t, 0)),
            pl.BlockSpec((1, T), lambda t, h: (0, 0)),
            pl.BlockSpec((1, HD), lambda t, h: (0, 0)),
        ],
        out_specs=pl.BlockSpec((1, TA, HD), lambda t, h: (h, t, 0)),
        out_shape=jax.ShapeDtypeStruct((H, T, HD), jnp.float32),
        scratch_shapes=[
            pltpu.VMEM((TA, T), jnp.float32),
            pltpu.VMEM((H, T, 2 * HD), jnp.bfloat16),
        ],
    )(qkv, qkv, qkv, fr_col, fr_row, attn_bias)


def _post_body(ao_ref, wo_ref, bo_ref, src_ref, g1_ref, be1_ref,
               gw_ref, gb_ref, x_ref, gl_ref):
    acc = jnp.dot(ao_ref[...], wo_ref[...], preferred_element_type=jnp.float32)
    x = _ln(src_ref[...] + acc + bo_ref[...], g1_ref[...], be1_ref[...])
    x_ref[...] = x
    gl_ref[...] = jnp.dot(x, gw_ref[...],
                          preferred_element_type=jnp.float32) + gb_ref[...]


def _post_attn(ao, wo, bo, src2, g1, be1, gw, gb):
    return pl.pallas_call(
        _post_body,
        grid=(T // TQ,),
        in_specs=[
            pl.BlockSpec((TQ, D), lambda t: (t, 0)),
            pl.BlockSpec((D, D), lambda t: (0, 0)),
            pl.BlockSpec((1, D), lambda t: (0, 0)),
            pl.BlockSpec((TQ, D), lambda t: (t, 0)),
            pl.BlockSpec((1, D), lambda t: (0, 0)),
            pl.BlockSpec((1, D), lambda t: (0, 0)),
            pl.BlockSpec((D, E), lambda t: (0, 0)),
            pl.BlockSpec((1, E), lambda t: (0, 0)),
        ],
        out_specs=[
            pl.BlockSpec((TQ, D), lambda t: (t, 0)),
            pl.BlockSpec((TQ, E), lambda t: (t, 0)),
        ],
        out_shape=[
            jax.ShapeDtypeStruct((T, D), jnp.float32),
            jax.ShapeDtypeStruct((T, E), jnp.float32),
        ],
    )(ao, wo, bo, src2, g1, be1, gw, gb)


def _route_body(gl_ref, d0_ref, d1_ref, w_ref, bexp_ref, nused_ref):
    gl = gl_ref[...]
    m = jnp.max(gl, axis=1, keepdims=True)
    p = jnp.exp(gl - m)
    g = p / jnp.sum(p, axis=1, keepdims=True)
    ioe = lax.broadcasted_iota(jnp.int32, (T, E), 1)
    m1 = jnp.max(g, axis=1, keepdims=True)
    a1 = jnp.min(jnp.where(g == m1, ioe, E), axis=1, keepdims=True)
    oh1 = (ioe == a1).astype(jnp.float32)
    gm = jnp.where(ioe == a1, -jnp.inf, g)
    m2 = jnp.max(gm, axis=1, keepdims=True)
    a2 = jnp.min(jnp.where(gm == m2, ioe, E), axis=1, keepdims=True)
    oh2 = (ioe == a2).astype(jnp.float32)
    w_ref[...] = jnp.concatenate([m1, m2], axis=1)

    c0 = oh1
    c1 = oh2
    k = 1
    while k < T:
        z = jnp.zeros((k, E), jnp.float32)
        c0 = c0 + jnp.concatenate([z, c0[:-k]], axis=0)
        c1 = c1 + jnp.concatenate([z, c1[:-k]], axis=0)
        k *= 2
    tot = jnp.sum(oh1 + oh2, axis=0, keepdims=True)
    pc = jnp.floor((tot + (BM - 1)) / BM)
    ci = pc
    for k in (1, 2, 4):
        ci = ci + jnp.concatenate(
            [jnp.zeros((1, k), jnp.float32), ci[:, :-k]], axis=1)
    bstart = ci - pc
    sstart = bstart * BM
    nb_used = jnp.sum(pc)

    before0 = (c0 - oh1) + (c1 - oh2)
    before1 = c0 + (c1 - oh2)
    d0 = jnp.sum(oh1 * (sstart + before0), axis=1, keepdims=True)
    d1 = jnp.sum(oh2 * (sstart + before1), axis=1, keepdims=True)
    d0_ref[...] = d0.astype(jnp.int32)
    d1_ref[...] = d1.astype(jnp.int32)

    bi = lax.broadcasted_iota(jnp.int32, (NB, E), 0).astype(jnp.float32)
    ind = jnp.logical_and(bi >= bstart, bi < bstart + pc).astype(jnp.float32)
    ev = lax.broadcasted_iota(jnp.int32, (NB, E), 1).astype(jnp.float32)
    eb = jnp.sum(ind * ev, axis=1, keepdims=True)
    last_e = jnp.max(eb)
    bi0 = lax.broadcasted_iota(jnp.int32, (NB, 1), 0).astype(jnp.float32)
    eb = jnp.where(bi0 < nb_used, eb, last_e)
    bexp_ref[...] = eb.astype(jnp.int32)
    nused_ref[...] = jnp.full((1, 1), nb_used, jnp.float32).astype(jnp.int32)


def _route(gl):
    return pl.pallas_call(
        _route_body,
        out_shape=[
            jax.ShapeDtypeStruct((T, 1), jnp.int32),
            jax.ShapeDtypeStruct((T, 1), jnp.int32),
            jax.ShapeDtypeStruct((T, 2), jnp.float32),
            jax.ShapeDtypeStruct((NB, 1), jnp.int32),
            jax.ShapeDtypeStruct((1, 1), jnp.int32),
        ],
    )(gl)


def _sc_scatter(x, d0, d1):
    mesh = plsc.VectorSubcoreMesh(core_axis_name="c", subcore_axis_name="s")

    @functools.partial(
        pl.kernel, mesh=mesh,
        out_type=jax.ShapeDtypeStruct((NS, D), jnp.float32),
        scratch_types=[
            pltpu.VMEM((CHUNK,), jnp.int32),
            pltpu.VMEM((CHUNK, D), jnp.float32),
            pltpu.SemaphoreType.DMA,
        ],
    )
    def k(x_hbm, d0_hbm, d1_hbm, out_hbm, idx_v, rows_v, sem):
        wid = lax.axis_index("s") * 2 + lax.axis_index("c")
        base = wid * CHUNK
        pltpu.sync_copy(x_hbm.at[pl.ds(base, CHUNK)], rows_v)
        pltpu.sync_copy(d0_hbm.at[pl.ds(base, CHUNK)], idx_v)
        pltpu.async_copy(rows_v, out_hbm.at[idx_v], sem).wait()
        pltpu.sync_copy(d1_hbm.at[pl.ds(base, CHUNK)], idx_v)
        pltpu.async_copy(rows_v, out_hbm.at[idx_v], sem).wait()

    return k(x, d0, d1)


def _ffn_body(be_ref, nu_ref, xs_ref, w1_ref, b1_ref, w2_ref, b2_ref, ys_ref):
    b = pl.program_id(0)

    @pl.when(b < nu_ref[0])
    def _():
        h = jnp.dot(xs_ref[...], w1_ref[0],
                    preferred_element_type=jnp.float32) + b1_ref[0]
        h = jnp.maximum(h, 0.0)
        ys_ref[...] = jnp.dot(h, w2_ref[0],
                              preferred_element_type=jnp.float32) + b2_ref[0]


def _ffn(bexp, nused, xs, w1, b1, w2, b2):
    grid_spec = pltpu.PrefetchScalarGridSpec(
        num_scalar_prefetch=2,
        grid=(NB,),
        in_specs=[
            pl.BlockSpec((BM, D),
                         lambda b, be, nu: (jnp.minimum(b, nu[0] - 1), 0)),
            pl.BlockSpec((1, D, FF), lambda b, be, nu: (be[b], 0, 0)),
            pl.BlockSpec((1, 1, FF), lambda b, be, nu: (be[b], 0, 0)),
            pl.BlockSpec((1, FF, D), lambda b, be, nu: (be[b], 0, 0)),
            pl.BlockSpec((1, 1, D), lambda b, be, nu: (be[b], 0, 0)),
        ],
        out_specs=pl.BlockSpec((BM, D),
                               lambda b, be, nu: (jnp.minimum(b, nu[0] - 1), 0)),
    )
    return pl.pallas_call(
        _ffn_body,
        grid_spec=grid_spec,
        out_shape=jax.ShapeDtypeStruct((NS, D), jnp.float32),
    )(bexp, nused, xs, w1, b1, w2, b2)


def _sc_gather(ys, d0, d1):
    mesh = plsc.VectorSubcoreMesh(core_axis_name="c", subcore_axis_name="s")

    @functools.partial(
        pl.kernel, mesh=mesh,
        out_type=(jax.ShapeDtypeStruct((T, D), jnp.float32),
                  jax.ShapeDtypeStruct((T, D), jnp.float32)),
        scratch_types=[
            pltpu.VMEM((CHUNK,), jnp.int32),
            pltpu.VMEM((CHUNK, D), jnp.float32),
            pltpu.SemaphoreType.DMA,
        ],
    )
    def k(ys_hbm, d0_hbm, d1_hbm, y0_hbm, y1_hbm, idx_v, rows_v, sem):
        wid = lax.axis_index("s") * 2 + lax.axis_index("c")
        base = wid * CHUNK
        pltpu.sync_copy(d0_hbm.at[pl.ds(base, CHUNK)], idx_v)
        pltpu.async_copy(ys_hbm.at[idx_v], rows_v, sem).wait()
        pltpu.sync_copy(rows_v, y0_hbm.at[pl.ds(base, CHUNK)])
        pltpu.sync_copy(d1_hbm.at[pl.ds(base, CHUNK)], idx_v)
        pltpu.async_copy(ys_hbm.at[idx_v], rows_v, sem).wait()
        pltpu.sync_copy(rows_v, y1_hbm.at[pl.ds(base, CHUNK)])

    return k(ys, d0, d1)


def _comb_body(x_ref, y0_ref, y1_ref, w_ref, g2_ref, be2_ref, o_ref):
    w = w_ref[...]
    ff = w[:, 0:1] * y0_ref[...] + w[:, 1:2] * y1_ref[...]
    o_ref[...] = _ln(x_ref[...] + ff, g2_ref[...], be2_ref[...])


def _combine(x, y0, y1, w, g2, be2):
    return pl.pallas_call(
        _comb_body,
        grid=(T // TQ,),
        in_specs=[
            pl.BlockSpec((TQ, D), lambda t: (t, 0)),
            pl.BlockSpec((TQ, D), lambda t: (t, 0)),
            pl.BlockSpec((TQ, D), lambda t: (t, 0)),
            pl.BlockSpec((TQ, 2), lambda t: (t, 0)),
            pl.BlockSpec((1, D), lambda t: (0, 0)),
            pl.BlockSpec((1, D), lambda t: (0, 0)),
        ],
        out_specs=pl.BlockSpec((TQ, D), lambda t: (t, 0)),
        out_shape=jax.ShapeDtypeStruct((T, D), jnp.float32),
    )(x, y0, y1, w, g2, be2)


def kernel(src, frac, Wq, bq, Wk, bk, Wv, bv, attn_bias, Wo, bo,
           gate_w, gate_b, W1, b1, W2, b2, g1, be1, g2, be2):
    src2 = src[0]
    fr = frac[0]
    csc = (HD ** -0.5) * 1.4426950408889634
    w3 = jnp.concatenate([Wq * csc, Wk, Wv], axis=1)
    b3 = jnp.concatenate([bq * csc, bk, bv]).reshape(1, 3 * D)

    qkv = _qkv(src2, w3, b3)
    ao = _attn(qkv, fr.reshape(T, 1), fr.reshape(1, T),
               attn_bias.reshape(1, HD))
    aot = ao.transpose(1, 0, 2).reshape(T, D)
    x, gl = _post_attn(aot, Wo, bo.reshape(1, D), src2,
                       g1.reshape(1, D), be1.reshape(1, D),
                       gate_w, gate_b.reshape(1, E))
    d0c, d1c, w2s, bexp, nused = _route(gl)
    d0 = d0c.reshape(T)
    d1 = d1c.reshape(T)
    xs = _sc_scatter(x, d0, d1)
    ys = _ffn(bexp.reshape(NB), nused.reshape(1), xs,
              W1, b1.reshape(E, 1, FF), W2, b2.reshape(E, 1, D))
    y0, y1 = _sc_gather(ys, d0, d1)
    y = _combine(x, y0, y1, w2s, g2.reshape(1, D), be2.reshape(1, D))
    return y.reshape(1, T, D)
